# baseline (speedup 1.0000x reference)
import functools

import jax
import jax.numpy as jnp
from jax import lax
from jax.experimental import pallas as pl
from jax.experimental.pallas import tpu as pltpu
from jax.experimental.pallas import tpu_sc as plsc


def _topk_body(x1_ref, x2_ref, i3_ref, i4_ref, s1_ref, s2_ref, *, nsteps):
    k = pl.program_id(0)
    p1 = jnp.sum(x1_ref[...], axis=(2, 3))
    p2 = jnp.sum(x2_ref[...], axis=(2, 3))

    @pl.when(k == 0)
    def _init():
        s1_ref[...] = p1
        s2_ref[...] = p2

    @pl.when(k > 0)
    def _acc():
        s1_ref[...] += p1
        s2_ref[...] += p2

    @pl.when(k == nsteps - 1)
    def _rank():
        B, C = s1_ref.shape
        half = C // 2
        ii = lax.broadcasted_iota(jnp.int32, (B, C, C), 1)
        jj = lax.broadcasted_iota(jnp.int32, (B, C, C), 2)
        boff = lax.broadcasted_iota(jnp.int32, (B, half), 0) * C

        def perm_of(m):
            mi = m[:, :, None]
            mj = m[:, None, :]
            hit = (mj > mi) | ((mj == mi) & (jj < ii))
            rank = jnp.sum(hit.astype(jnp.int32), axis=2)
            er = rank[:, :, None] == jj
            perm = jnp.sum(jnp.where(er, ii, 0), axis=1)
            return perm[:, :half] + boff

        i3_ref[...] = perm_of(s1_ref[...])
        i4_ref[...] = perm_of(s2_ref[...])


def _topk_rows(x1, x2):
    B, C, H, W = x1.shape
    half = C // 2
    hchunk = 16
    nsteps = H // hchunk
    return pl.pallas_call(
        functools.partial(_topk_body, nsteps=nsteps),
        grid=(nsteps,),
        in_specs=[
            pl.BlockSpec((B, C, hchunk, W), lambda k: (0, 0, k, 0)),
            pl.BlockSpec((B, C, hchunk, W), lambda k: (0, 0, k, 0)),
        ],
        out_specs=[
            pl.BlockSpec((B, half), lambda k: (0, 0)),
            pl.BlockSpec((B, half), lambda k: (0, 0)),
        ],
        out_shape=[
            jax.ShapeDtypeStruct((B, half), jnp.int32),
            jax.ShapeDtypeStruct((B, half), jnp.int32),
        ],
        scratch_shapes=[
            pltpu.VMEM((B, C), jnp.float32),
            pltpu.VMEM((B, C), jnp.float32),
        ],
    )(x1, x2)


def kernel(x1, x2):
    B, C, H, W = x1.shape
    half = C // 2
    i3, i4 = _topk_rows(x1, x2)

    i3f = i3.reshape(B * half)
    i4f = i4.reshape(B * half)
    x1v = x1.reshape(B * C, H, W)
    x2v = x2.reshape(B * C, H, W)

    nworkers = 32
    rpw = (B * C) // nworkers
    nw_half = nworkers // 2

    mesh = plsc.VectorSubcoreMesh(core_axis_name="c", subcore_axis_name="s")

    @functools.partial(
        pl.kernel,
        mesh=mesh,
        out_type=jax.ShapeDtypeStruct((B * C, H, W), jnp.float32),
        scratch_types=[
            pltpu.VMEM((rpw,), jnp.int32),
            pltpu.SemaphoreType.DMA,
        ],
    )
    def sc_gather(x1_hbm, x2_hbm, i3_hbm, i4_hbm, out_hbm, idx_vmem, sem):
        cid = lax.axis_index("c")
        sid = lax.axis_index("s")
        wid = sid * 2 + cid

        def do_half(x_hbm, idx_hbm, base, j0):
            pltpu.sync_copy(idx_hbm.at[pl.ds(j0, rpw)], idx_vmem)
            bb = j0 // half
            r0 = bb * C + base + (j0 - bb * half)
            v0 = idx_vmem[pl.ds(0, 16)]
            v1 = idx_vmem[pl.ds(rpw - 16, 16)]

            def row_at(i):
                return v0[i] if i < 16 else v1[i - (rpw - 16)]

            window = 8
            inflight = []
            for i in range(rpw):
                if len(inflight) == window:
                    inflight.pop(0).wait()
                inflight.append(
                    pltpu.async_copy(
                        x_hbm.at[pl.ds(row_at(i), 1)],
                        out_hbm.at[pl.ds(r0 + i, 1)],
                        sem,
                    )
                )
            for hnd in inflight:
                hnd.wait()

        @pl.when(wid < nw_half)
        def _h0():
            do_half(x1_hbm, i3_hbm, 0, wid * rpw)

        @pl.when(wid >= nw_half)
        def _h1():
            do_half(x2_hbm, i4_hbm, half, (wid - nw_half) * rpw)

    out = sc_gather(x1v, x2v, i3f, i4f)
    return out.reshape(B, C, H, W)


# speedup vs baseline: 8.5480x; 8.5480x over previous
import functools

import jax
import jax.numpy as jnp
from jax import lax
from jax.experimental import pallas as pl
from jax.experimental.pallas import tpu as pltpu
from jax.experimental.pallas import tpu_sc as plsc


def _topk_body(x1_ref, x2_ref, i3_ref, i4_ref, s1_ref, s2_ref, *, nsteps):
    k = pl.program_id(0)
    p1 = jnp.sum(x1_ref[...], axis=(2, 3))
    p2 = jnp.sum(x2_ref[...], axis=(2, 3))

    @pl.when(k == 0)
    def _init():
        s1_ref[...] = p1
        s2_ref[...] = p2

    @pl.when(k > 0)
    def _acc():
        s1_ref[...] += p1
        s2_ref[...] += p2

    @pl.when(k == nsteps - 1)
    def _rank():
        B, C = s1_ref.shape
        half = C // 2
        ii = lax.broadcasted_iota(jnp.int32, (B, C, C), 1)
        jj = lax.broadcasted_iota(jnp.int32, (B, C, C), 2)
        boff = lax.broadcasted_iota(jnp.int32, (B, half), 0) * C

        def perm_of(m):
            mi = m[:, :, None]
            mj = m[:, None, :]
            hit = (mj > mi) | ((mj == mi) & (jj < ii))
            rank = jnp.sum(hit.astype(jnp.int32), axis=2)
            er = rank[:, :, None] == jj
            perm = jnp.sum(jnp.where(er, ii, 0), axis=1)
            return perm[:, :half] + boff

        i3_ref[...] = perm_of(s1_ref[...])
        i4_ref[...] = perm_of(s2_ref[...])


def _topk_rows(x1, x2):
    B, C, H, W = x1.shape
    half = C // 2
    hchunk = 16
    nsteps = H // hchunk
    return pl.pallas_call(
        functools.partial(_topk_body, nsteps=nsteps),
        grid=(nsteps,),
        in_specs=[
            pl.BlockSpec((B, C, hchunk, W), lambda k: (0, 0, k, 0)),
            pl.BlockSpec((B, C, hchunk, W), lambda k: (0, 0, k, 0)),
        ],
        out_specs=[
            pl.BlockSpec((B, half), lambda k: (0, 0)),
            pl.BlockSpec((B, half), lambda k: (0, 0)),
        ],
        out_shape=[
            jax.ShapeDtypeStruct((B, half), jnp.int32),
            jax.ShapeDtypeStruct((B, half), jnp.int32),
        ],
        scratch_shapes=[
            pltpu.VMEM((B, C), jnp.float32),
            pltpu.VMEM((B, C), jnp.float32),
        ],
    )(x1, x2)


def kernel(x1, x2):
    B, C, H, W = x1.shape
    half = C // 2
    i3, i4 = _topk_rows(x1, x2)

    i3f = i3.reshape(B * half)
    i4f = i4.reshape(B * half)
    x1v = x1.reshape(B * C, H, W)
    x2v = x2.reshape(B * C, H, W)

    nworkers = 32
    rpw = (B * C) // nworkers
    nw_half = nworkers // 2

    mesh = plsc.VectorSubcoreMesh(core_axis_name="c", subcore_axis_name="s")

    @functools.partial(
        pl.kernel,
        mesh=mesh,
        out_type=jax.ShapeDtypeStruct((B * C, H, W), jnp.float32),
        scratch_types=[
            pltpu.VMEM((rpw,), jnp.int32),
            pltpu.VMEM((1, H, W), jnp.float32),
            pltpu.VMEM((1, H, W), jnp.float32),
            pltpu.SemaphoreType.DMA,
            pltpu.SemaphoreType.DMA,
        ],
    )
    def sc_gather(x1_hbm, x2_hbm, i3_hbm, i4_hbm, out_hbm,
                  idx_vmem, buf0, buf1, sem0, sem1):
        cid = lax.axis_index("c")
        sid = lax.axis_index("s")
        wid = sid * 2 + cid

        def do_half(x_hbm, idx_hbm, base, j0):
            pltpu.sync_copy(idx_hbm.at[pl.ds(j0, rpw)], idx_vmem)
            bb = j0 // half
            r0 = bb * C + base + (j0 - bb * half)
            v0 = idx_vmem[pl.ds(0, 16)]
            v1 = idx_vmem[pl.ds(rpw - 16, 16)]

            def row_at(i):
                return v0[i] if i < 16 else v1[i - (rpw - 16)]

            bufs = (buf0, buf1)
            sems = (sem0, sem1)
            handles = [
                pltpu.async_copy(x_hbm.at[pl.ds(row_at(0), 1)], bufs[0], sems[0])
            ]
            for i in range(rpw):
                handles[i].wait()
                if i + 1 < rpw:
                    handles.append(
                        pltpu.async_copy(
                            x_hbm.at[pl.ds(row_at(i + 1), 1)],
                            bufs[(i + 1) % 2],
                            sems[(i + 1) % 2],
                        )
                    )
                pltpu.sync_copy(bufs[i % 2], out_hbm.at[pl.ds(r0 + i, 1)])

        @pl.when(wid < nw_half)
        def _h0():
            do_half(x1_hbm, i3_hbm, 0, wid * rpw)

        @pl.when(wid >= nw_half)
        def _h1():
            do_half(x2_hbm, i4_hbm, half, (wid - nw_half) * rpw)

    out = sc_gather(x1v, x2v, i3f, i4f)
    return out.reshape(B, C, H, W)


# speedup vs baseline: 10.0685x; 1.1779x over previous
import functools

import jax
import jax.numpy as jnp
from jax import lax
from jax.experimental import pallas as pl
from jax.experimental.pallas import tpu as pltpu
from jax.experimental.pallas import tpu_sc as plsc


def _topk_body(m1_ref, m2_ref, i3_ref, i4_ref):
    B, C = m1_ref.shape
    half = C // 2
    ii = lax.broadcasted_iota(jnp.int32, (B, C, C), 1)
    jj = lax.broadcasted_iota(jnp.int32, (B, C, C), 2)
    boff = lax.broadcasted_iota(jnp.int32, (B, half), 0) * C

    def perm_of(m):
        mi = m[:, :, None]
        mj = m[:, None, :]
        hit = (mj > mi) | ((mj == mi) & (jj < ii))
        rank = jnp.sum(hit.astype(jnp.int32), axis=2)
        er = rank[:, :, None] == jj
        perm = jnp.sum(jnp.where(er, ii, 0), axis=1)
        return perm[:, :half] + boff

    i3_ref[...] = perm_of(m1_ref[...])
    i4_ref[...] = perm_of(m2_ref[...])


def kernel(x1, x2):
    B, C, H, W = x1.shape
    half = C // 2

    m1 = jnp.mean(x1, axis=(2, 3))
    m2 = jnp.mean(x2, axis=(2, 3))

    i3, i4 = pl.pallas_call(
        _topk_body,
        out_shape=[
            jax.ShapeDtypeStruct((B, half), jnp.int32),
            jax.ShapeDtypeStruct((B, half), jnp.int32),
        ],
    )(m1, m2)

    i3f = i3.reshape(B * half)
    i4f = i4.reshape(B * half)
    x1v = x1.reshape(B * C, H, W)
    x2v = x2.reshape(B * C, H, W)

    nworkers = 32
    rpw = (B * C) // nworkers
    nw_half = nworkers // 2

    mesh = plsc.VectorSubcoreMesh(core_axis_name="c", subcore_axis_name="s")

    @functools.partial(
        pl.kernel,
        mesh=mesh,
        out_type=jax.ShapeDtypeStruct((B * C, H, W), jnp.float32),
        scratch_types=[
            pltpu.VMEM((rpw,), jnp.int32),
            pltpu.VMEM((1, H, W), jnp.float32),
            pltpu.VMEM((1, H, W), jnp.float32),
            pltpu.SemaphoreType.DMA,
            pltpu.SemaphoreType.DMA,
        ],
    )
    def sc_gather(x1_hbm, x2_hbm, i3_hbm, i4_hbm, out_hbm,
                  idx_vmem, buf0, buf1, sem0, sem1):
        cid = lax.axis_index("c")
        sid = lax.axis_index("s")
        wid = sid * 2 + cid

        def do_half(x_hbm, idx_hbm, base, j0):
            pltpu.sync_copy(idx_hbm.at[pl.ds(j0, rpw)], idx_vmem)
            bb = j0 // half
            r0 = bb * C + base + (j0 - bb * half)
            v0 = idx_vmem[pl.ds(0, 16)]
            v1 = idx_vmem[pl.ds(rpw - 16, 16)]

            def row_at(i):
                return v0[i] if i < 16 else v1[i - (rpw - 16)]

            bufs = (buf0, buf1)
            sems = (sem0, sem1)
            handles = [
                pltpu.async_copy(x_hbm.at[pl.ds(row_at(0), 1)], bufs[0], sems[0])
            ]
            for i in range(rpw):
                handles[i].wait()
                if i + 1 < rpw:
                    handles.append(
                        pltpu.async_copy(
                            x_hbm.at[pl.ds(row_at(i + 1), 1)],
                            bufs[(i + 1) % 2],
                            sems[(i + 1) % 2],
                        )
                    )
                pltpu.sync_copy(bufs[i % 2], out_hbm.at[pl.ds(r0 + i, 1)])

        @pl.when(wid < nw_half)
        def _h0():
            do_half(x1_hbm, i3_hbm, 0, wid * rpw)

        @pl.when(wid >= nw_half)
        def _h1():
            do_half(x2_hbm, i4_hbm, half, (wid - nw_half) * rpw)

    out = sc_gather(x1v, x2v, i3f, i4f)
    return out.reshape(B, C, H, W)
